# baseline (speedup 1.0000x reference)
import jax
import jax.numpy as jnp
from jax.experimental import pallas as pl

N_TOKENS = 32768
D = 768
E = 8
K = 2
BT = 1024


def _router_block(x_ref, w_ref, idx_ref, topk_ref, probs_ref):
    x = x_ref[...]
    w = w_ref[...]
    logits_t = jax.lax.dot_general(
        w, x, (((1,), (1,)), ((), ())), preferred_element_type=jnp.float32)

    m = jnp.max(logits_t, axis=0, keepdims=True)
    ex = jnp.exp(logits_t - m)
    denom = jnp.sum(ex, axis=0, keepdims=True)
    probs_t = ex / denom

    row = jax.lax.broadcasted_iota(jnp.int32, (E, BT), 0)
    big = jnp.int32(E)
    p1 = jnp.max(probs_t, axis=0, keepdims=True)
    i1 = jnp.min(jnp.where(probs_t == p1, row, big), axis=0, keepdims=True)
    rest = jnp.where(row != i1, probs_t, -1.0)
    p2 = jnp.max(rest, axis=0, keepdims=True)
    i2 = jnp.min(jnp.where(rest == p2, row, big), axis=0, keepdims=True)

    rn = 1.0 / (p1 + p2 + 1e-9)

    probs_ref[...] = probs_t.T
    idx_ref[...] = jnp.concatenate([i1, i2], axis=0).T
    topk_ref[...] = jnp.concatenate([p1 * rn, p2 * rn], axis=0).T


@jax.jit
def kernel(x, W_gate, W_noisy):
    grid = (N_TOKENS // BT,)
    out_shapes = (
        jax.ShapeDtypeStruct((N_TOKENS, K), jnp.int32),
        jax.ShapeDtypeStruct((N_TOKENS, K), jnp.float32),
        jax.ShapeDtypeStruct((N_TOKENS, E), jnp.float32),
    )
    topk_idx, topk_probs, probs = pl.pallas_call(
        _router_block,
        grid=grid,
        in_specs=[
            pl.BlockSpec((BT, D), lambda i: (i, 0)),
            pl.BlockSpec((E, D), lambda i: (0, 0)),
        ],
        out_specs=(
            pl.BlockSpec((BT, K), lambda i: (i, 0)),
            pl.BlockSpec((BT, K), lambda i: (i, 0)),
            pl.BlockSpec((BT, E), lambda i: (i, 0)),
        ),
        out_shape=out_shapes,
    )(x, W_gate)
    return topk_idx, topk_probs, probs


# speedup vs baseline: 1.0875x; 1.0875x over previous
import jax
import jax.numpy as jnp
from jax.experimental import pallas as pl

N_TOKENS = 32768
D = 768
E = 8
K = 2
BT = 4096


def _router_block(x_ref, w_ref, idx_ref, topk_ref, probs_ref):
    x = x_ref[...]
    w = w_ref[...]
    logits_t = jax.lax.dot_general(
        w, x, (((1,), (1,)), ((), ())), preferred_element_type=jnp.float32)

    m = jnp.max(logits_t, axis=0, keepdims=True)
    ex = jnp.exp(logits_t - m)
    denom = jnp.sum(ex, axis=0, keepdims=True)
    probs_t = ex / denom

    row = jax.lax.broadcasted_iota(jnp.int32, (E, BT), 0)
    big = jnp.int32(E)
    p1 = jnp.max(probs_t, axis=0, keepdims=True)
    i1 = jnp.min(jnp.where(probs_t == p1, row, big), axis=0, keepdims=True)
    rest = jnp.where(row != i1, probs_t, -1.0)
    p2 = jnp.max(rest, axis=0, keepdims=True)
    i2 = jnp.min(jnp.where(rest == p2, row, big), axis=0, keepdims=True)

    rn = 1.0 / (p1 + p2 + 1e-9)

    probs_ref[...] = probs_t.T
    idx_ref[...] = jnp.concatenate([i1, i2], axis=0).T
    topk_ref[...] = jnp.concatenate([p1 * rn, p2 * rn], axis=0).T


@jax.jit
def kernel(x, W_gate, W_noisy):
    grid = (N_TOKENS // BT,)
    out_shapes = (
        jax.ShapeDtypeStruct((N_TOKENS, K), jnp.int32),
        jax.ShapeDtypeStruct((N_TOKENS, K), jnp.float32),
        jax.ShapeDtypeStruct((N_TOKENS, E), jnp.float32),
    )
    topk_idx, topk_probs, probs = pl.pallas_call(
        _router_block,
        grid=grid,
        in_specs=[
            pl.BlockSpec((BT, D), lambda i: (i, 0)),
            pl.BlockSpec((E, D), lambda i: (0, 0)),
        ],
        out_specs=(
            pl.BlockSpec((BT, K), lambda i: (i, 0)),
            pl.BlockSpec((BT, K), lambda i: (i, 0)),
            pl.BlockSpec((BT, E), lambda i: (i, 0)),
        ),
        out_shape=out_shapes,
    )(x, W_gate)
    return topk_idx, topk_probs, probs


# speedup vs baseline: 1.1101x; 1.0207x over previous
import jax
import jax.numpy as jnp
from jax.experimental import pallas as pl
from jax.experimental.pallas import tpu as pltpu

N_TOKENS = 32768
D = 768
E = 8
K = 2
CHUNK = 1024
NBUF = 6
NSTEP = N_TOKENS // CHUNK


def _compute(x, w, idx_ref, topk_ref, probs_ref):
    logits_t = jax.lax.dot_general(
        w, x, (((1,), (1,)), ((), ())), preferred_element_type=jnp.float32)

    m = jnp.max(logits_t, axis=0, keepdims=True)
    ex = jnp.exp(logits_t - m)
    denom = jnp.sum(ex, axis=0, keepdims=True)
    probs_t = ex / denom

    row = jax.lax.broadcasted_iota(jnp.int32, (E, CHUNK), 0)
    big = jnp.int32(E)
    p1 = jnp.max(probs_t, axis=0, keepdims=True)
    i1 = jnp.min(jnp.where(probs_t == p1, row, big), axis=0, keepdims=True)
    rest = jnp.where(row != i1, probs_t, -1.0)
    p2 = jnp.max(rest, axis=0, keepdims=True)
    i2 = jnp.min(jnp.where(rest == p2, row, big), axis=0, keepdims=True)

    rn = 1.0 / (p1 + p2 + 1e-9)

    probs_ref[...] = probs_t.T
    idx_ref[...] = jnp.concatenate([i1, i2], axis=0).T
    topk_ref[...] = jnp.concatenate([p1 * rn, p2 * rn], axis=0).T


def _body(x_hbm, w_ref, idx_ref, topk_ref, probs_ref, bufs, sems):
    i = pl.program_id(0)

    def copy(c, slot):
        pltpu.make_async_copy(
            x_hbm.at[pl.ds(c * CHUNK, CHUNK), :],
            bufs.at[slot],
            sems.at[slot],
        ).start()

    @pl.when(i == 0)
    def _prime():
        for c in range(NBUF):
            copy(c, c)

    slot = jax.lax.rem(i, NBUF)
    pltpu.make_async_copy(
        x_hbm.at[pl.ds(0, CHUNK), :], bufs.at[slot], sems.at[slot]
    ).wait()

    x = bufs[slot]
    w = w_ref[...]
    _compute(x, w, idx_ref, topk_ref, probs_ref)

    nxt = i + NBUF

    @pl.when(nxt < NSTEP)
    def _refill():
        copy(nxt, slot)


@jax.jit
def kernel(x, W_gate, W_noisy):
    out_shapes = (
        jax.ShapeDtypeStruct((N_TOKENS, K), jnp.int32),
        jax.ShapeDtypeStruct((N_TOKENS, K), jnp.float32),
        jax.ShapeDtypeStruct((N_TOKENS, E), jnp.float32),
    )
    topk_idx, topk_probs, probs = pl.pallas_call(
        _body,
        grid=(NSTEP,),
        in_specs=[
            pl.BlockSpec(memory_space=pl.ANY),
            pl.BlockSpec((E, D), lambda i: (0, 0)),
        ],
        out_specs=(
            pl.BlockSpec((CHUNK, K), lambda i: (i, 0)),
            pl.BlockSpec((CHUNK, K), lambda i: (i, 0)),
            pl.BlockSpec((CHUNK, E), lambda i: (i, 0)),
        ),
        out_shape=out_shapes,
        scratch_shapes=[
            pltpu.VMEM((NBUF, CHUNK, D), jnp.float32),
            pltpu.SemaphoreType.DMA((NBUF,)),
        ],
    )(x, W_gate)
    return topk_idx, topk_probs, probs


# speedup vs baseline: 2.3653x; 2.1308x over previous
import jax
import jax.numpy as jnp
from jax.experimental import pallas as pl

N_TOKENS = 32768
D = 768
E = 8
K = 2
BT = 1024
NSLICE = 8
ROWS = BT * NSLICE


def _router_slice(x, w, s, idx_ref, topk_ref, probs_ref):
    logits_t = jax.lax.dot_general(
        w, x, (((1,), (1,)), ((), ())), preferred_element_type=jnp.float32)

    m = jnp.max(logits_t, axis=0, keepdims=True)
    ex = jnp.exp(logits_t - m)
    denom = jnp.sum(ex, axis=0, keepdims=True)
    probs_t = ex / denom

    row = jax.lax.broadcasted_iota(jnp.int32, (E, BT), 0)
    big = jnp.int32(E)
    p1 = jnp.max(probs_t, axis=0, keepdims=True)
    i1 = jnp.min(jnp.where(probs_t == p1, row, big), axis=0, keepdims=True)
    rest = jnp.where(row != i1, probs_t, -1.0)
    p2 = jnp.max(rest, axis=0, keepdims=True)
    i2 = jnp.min(jnp.where(rest == p2, row, big), axis=0, keepdims=True)

    rn = 1.0 / (p1 + p2 + 1e-9)

    sl = pl.ds(s * BT, BT)
    probs_ref[:, sl] = probs_t
    idx_ref[:, sl] = jnp.concatenate([i1, i2], axis=0)
    topk_ref[:, sl] = jnp.concatenate([p1 * rn, p2 * rn], axis=0)


def _body(*refs):
    xs = refs[:NSLICE]
    w_ref = refs[NSLICE]
    idx_ref, topk_ref, probs_ref = refs[NSLICE + 1:]
    w = w_ref[...]
    for s in range(NSLICE):
        _router_slice(xs[s][...], w, s, idx_ref, topk_ref, probs_ref)


@jax.jit
def kernel(x, W_gate, W_noisy):
    grid = (N_TOKENS // ROWS,)
    out_shapes = (
        jax.ShapeDtypeStruct((K, N_TOKENS), jnp.int32),
        jax.ShapeDtypeStruct((K, N_TOKENS), jnp.float32),
        jax.ShapeDtypeStruct((E, N_TOKENS), jnp.float32),
    )
    in_specs = [
        pl.BlockSpec((BT, D), (lambda i, s=s: (i * NSLICE + s, 0)))
        for s in range(NSLICE)
    ] + [pl.BlockSpec((E, D), lambda i: (0, 0))]
    topk_idx, topk_probs, probs = pl.pallas_call(
        _body,
        grid=grid,
        in_specs=in_specs,
        out_specs=(
            pl.BlockSpec((K, ROWS), lambda i: (0, i)),
            pl.BlockSpec((K, ROWS), lambda i: (0, i)),
            pl.BlockSpec((E, ROWS), lambda i: (0, i)),
        ),
        out_shape=out_shapes,
    )(*([x] * NSLICE), W_gate)
    return topk_idx.T, topk_probs.T, probs.T
